# baseline (speedup 1.0000x reference)
import functools

import jax
import jax.numpy as jnp
from jax import lax
from jax.experimental import pallas as pl
from jax.experimental.pallas import tpu as pltpu
from jax.experimental.pallas import tpu_sc as plsc

NUM_E = 1000000
H_DIM = 64
BATCH = 16384

_NC = 2
_NS = 16
_L = 16
_NW = _NC * _NS
_BPW = BATCH // _NW

_mesh = plsc.VectorSubcoreMesh(core_axis_name="c", subcore_axis_name="s")


@functools.partial(
    pl.kernel,
    mesh=_mesh,
    out_type=jax.ShapeDtypeStruct((BATCH, H_DIM), jnp.float32),
    scratch_types=[
        pltpu.VMEM((_BPW,), jnp.int32),
        pltpu.VMEM((_BPW, H_DIM), jnp.float32),
        pltpu.SemaphoreType.DMA,
    ],
    compiler_params=pltpu.CompilerParams(use_tc_tiling_on_sc=False),
)
def _gate_sc(table_hbm, idx_hbm, out_hbm, idx_v, rows_v, sem):
    wid = lax.axis_index("s") * _NC + lax.axis_index("c")
    base = wid * _BPW
    pltpu.sync_copy(idx_hbm.at[pl.ds(base, _BPW)], idx_v)
    pltpu.async_copy(table_hbm.at[idx_v], rows_v, sem).wait()

    def body(r, carry):
        for c in range(H_DIM // _L):
            x = rows_v[r, pl.ds(c * _L, _L)]
            rows_v[r, pl.ds(c * _L, _L)] = 1.0 / (1.0 + jnp.exp(-x))
        return carry

    lax.fori_loop(0, _BPW, body, 0, unroll=4)
    pltpu.sync_copy(rows_v, out_hbm.at[pl.ds(base, _BPW)])


def kernel(X, Y, gate_theta):
    del X
    return _gate_sc(gate_theta, Y.astype(jnp.int32))
